# Initial kernel scaffold; baseline (speedup 1.0000x reference)
import jax
import jax.numpy as jnp
from jax.experimental import pallas as pl


def kernel(h, edge_index, edge_attr, ew0, ew1, ew2, ew3, ew4, ew5, ew6, ew7, nw0, nw1, nw2, nw3, nw4, nw5, nw6, nw7):
    raise NotImplementedError("write your pallas kernel here")


# speedup vs baseline: 2.3043x; 2.3043x over previous
import functools

import jax
import jax.numpy as jnp
from jax import lax
from jax.experimental import pallas as pl
from jax.experimental.pallas import tpu as pltpu
from jax.experimental.pallas import tpu_sc as plsc

NC = 2
NS = 16
NW = NC * NS
LANES = 16
CH = 128


def _relu(x):
    return jnp.maximum(x, 0.0)


def _tc_ab(h, wa, wb):
    def body(h_ref, wa_ref, wb_ref, a_ref, b_ref):
        hv = h_ref[...]
        a_ref[...] = jnp.dot(hv, wa_ref[...], preferred_element_type=jnp.float32)
        b_ref[...] = jnp.dot(hv, wb_ref[...], preferred_element_type=jnp.float32)

    n, d = h.shape
    return pl.pallas_call(
        body,
        out_shape=(
            jax.ShapeDtypeStruct((n, wa.shape[1]), jnp.float32),
            jax.ShapeDtypeStruct((n, wb.shape[1]), jnp.float32),
        ),
    )(h, wa, wb)


def _sc_gather_sum(a, b, row, col):
    n, d = a.shape
    e = row.shape[0]
    ew = e // NW
    n_full = ew // CH
    rem = ew % CH
    mesh = plsc.VectorSubcoreMesh(
        core_axis_name="c", subcore_axis_name="s", num_cores=NC, num_subcores=NS
    )

    scratch = [
        pltpu.VMEM((CH,), jnp.int32),
        pltpu.VMEM((CH,), jnp.int32),
        pltpu.VMEM((CH, d), jnp.float32),
        pltpu.VMEM((CH, d), jnp.float32),
        pltpu.SemaphoreType.DMA,
        pltpu.SemaphoreType.DMA,
    ]
    if rem:
        scratch += [
            pltpu.VMEM((rem,), jnp.int32),
            pltpu.VMEM((rem,), jnp.int32),
            pltpu.VMEM((rem, d), jnp.float32),
            pltpu.VMEM((rem, d), jnp.float32),
        ]

    @functools.partial(
        pl.kernel,
        out_type=jax.ShapeDtypeStruct((e, d), jnp.float32),
        mesh=mesh,
        scratch_types=scratch,
    )
    def gather_sum(a_hbm, b_hbm, row_hbm, col_hbm, g_hbm, *scr):
        if rem:
            idxr, idxc, bufa, bufb, sem1, sem2, idxr2, idxc2, bufa2, bufb2 = scr
        else:
            idxr, idxc, bufa, bufb, sem1, sem2 = scr
        wid = lax.axis_index("c") * NS + lax.axis_index("s")
        base = wid * ew

        def do_chunk(off, ch, ir, ic, ba, bb):
            pltpu.sync_copy(row_hbm.at[pl.ds(off, ch)], ir)
            pltpu.sync_copy(col_hbm.at[pl.ds(off, ch)], ic)
            cpa = pltpu.async_copy(a_hbm.at[ir], ba, sem1)
            cpb = pltpu.async_copy(b_hbm.at[ic], bb, sem2)
            cpa.wait()
            cpb.wait()

            def add_row(j, carry):
                for cc in range(d // LANES):
                    sl = pl.ds(cc * LANES, LANES)
                    ba[j, sl] = ba[j, sl] + bb[j, sl]
                return carry

            lax.fori_loop(0, ch, add_row, 0)
            pltpu.sync_copy(ba, g_hbm.at[pl.ds(off, ch)])

        if n_full:
            def step(t, carry):
                do_chunk(base + t * CH, CH, idxr, idxc, bufa, bufb)
                return carry

            lax.fori_loop(0, n_full, step, 0)
        if rem:
            do_chunk(base + n_full * CH, rem, idxr2, idxc2, bufa2, bufb2)

    return gather_sum(a, b, row, col)


def _tc_edge_mlp(g, ea, w0c, b0, w1, b1, w2, b2, w3, b3, block_e):
    e, d = g.shape
    de = ea.shape[1]
    grid = (e // block_e,)

    def body(g_ref, ea_ref, w0c_ref, b0_ref, w1_ref, b1_ref, w2_ref, b2_ref,
             w3_ref, b3_ref, out_ref):
        x = g_ref[...] + jnp.dot(ea_ref[...], w0c_ref[...],
                                 preferred_element_type=jnp.float32) + b0_ref[...]
        x = _relu(x)
        x = _relu(jnp.dot(x, w1_ref[...], preferred_element_type=jnp.float32)
                  + b1_ref[...])
        x = _relu(jnp.dot(x, w2_ref[...], preferred_element_type=jnp.float32)
                  + b2_ref[...])
        out_ref[...] = jnp.dot(x, w3_ref[...],
                               preferred_element_type=jnp.float32) + b3_ref[...]

    full = lambda shape: pl.BlockSpec(shape, lambda i: (0,) * len(shape))
    return pl.pallas_call(
        body,
        grid=grid,
        in_specs=[
            pl.BlockSpec((block_e, d), lambda i: (i, 0)),
            pl.BlockSpec((block_e, de), lambda i: (i, 0)),
            full(w0c.shape), full(b0.shape), full(w1.shape), full(b1.shape),
            full(w2.shape), full(b2.shape), full(w3.shape), full(b3.shape),
        ],
        out_specs=pl.BlockSpec((block_e, de), lambda i: (i, 0)),
        out_shape=jax.ShapeDtypeStruct((e, de), jnp.float32),
    )(g, ea, w0c, b0, w1, b1, w2, b2, w3, b3)


def _sc_scatter_add(ev, col, n):
    e, de = ev.shape
    ew = e // NW
    n_full = ew // CH
    rem = ew % CH
    half = n // 2
    mesh = plsc.VectorSubcoreMesh(
        core_axis_name="c", subcore_axis_name="s", num_cores=NC, num_subcores=NS
    )

    scratch = [
        pltpu.VMEM((CH,), jnp.int32),
        pltpu.VMEM((CH, de), jnp.float32),
        pltpu.VMEM((half * de + de,), jnp.float32),
    ]
    if rem:
        scratch += [
            pltpu.VMEM((rem,), jnp.int32),
            pltpu.VMEM((rem, de), jnp.float32),
        ]

    @functools.partial(
        pl.kernel,
        out_type=jax.ShapeDtypeStruct((NW * 2 * half * de,), jnp.float32),
        mesh=mesh,
        scratch_types=scratch,
    )
    def scatter(e_hbm, col_hbm, out_hbm, *scr):
        if rem:
            idxv, ebuf, acc, idxv2, ebuf2 = scr
        else:
            idxv, ebuf, acc = scr
        c = lax.axis_index("c")
        s = lax.axis_index("s")
        wid = c * NS + s
        base = wid * ew
        lanes = lax.iota(jnp.int32, de)

        for p in range(2):
            lo = p * half

            def zrow(j, carry):
                acc[pl.ds(j * de, de)] = jnp.zeros((de,), jnp.float32)
                return carry

            lax.fori_loop(0, half, zrow, 0)

            def do_chunk(off, ch, iv, eb):
                pltpu.sync_copy(col_hbm.at[pl.ds(off, ch)], iv)
                pltpu.sync_copy(e_hbm.at[pl.ds(off, ch)], eb)

                def group(g, carry):
                    colvec = iv[pl.ds(g * LANES, LANES)] * de
                    for l in range(LANES):
                        el = colvec[l] - lo * de
                        ok = jnp.logical_and(el >= 0, el < half * de)
                        off_el = jnp.where(ok, el, half * de)
                        sl = pl.ds(off_el, de)
                        acc[sl] = acc[sl] + eb[g * LANES + l, :]
                    return carry

                lax.fori_loop(0, ch // LANES, group, 0)

            if n_full:
                def step(t, carry):
                    do_chunk(base + t * CH, CH, idxv, ebuf)
                    return carry

                lax.fori_loop(0, n_full, step, 0)
            if rem:
                do_chunk(base + n_full * CH, rem, idxv2, ebuf2)

            pltpu.sync_copy(
                acc.at[pl.ds(0, half * de)],
                out_hbm.at[pl.ds((wid * 2 + p) * half * de, half * de)],
            )

    return scatter(ev, col), half


def _tc_reduce_partials(partials, block):
    nw, tot = partials.shape

    def body(p_ref, out_ref):
        out_ref[...] = jnp.sum(p_ref[...], axis=0)

    return pl.pallas_call(
        body,
        out_shape=jax.ShapeDtypeStruct((tot,), jnp.float32),
    )(partials)


def _tc_node_mlp(h, msg, na, nb, b0, w1, b1, w2, b2, w3, b3, block_n):
    n, d = h.shape
    de = msg.shape[1]
    grid = (n // block_n,)

    def body(h_ref, msg_ref, na_ref, nb_ref, b0_ref, w1_ref, b1_ref,
             w2_ref, b2_ref, w3_ref, b3_ref, out_ref):
        msg = msg_ref[...]
        x = (jnp.dot(h_ref[...], na_ref[...], preferred_element_type=jnp.float32)
             + jnp.dot(msg, nb_ref[...], preferred_element_type=jnp.float32)
             + b0_ref[...])
        x = _relu(x)
        x = _relu(jnp.dot(x, w1_ref[...], preferred_element_type=jnp.float32)
                  + b1_ref[...])
        x = _relu(jnp.dot(x, w2_ref[...], preferred_element_type=jnp.float32)
                  + b2_ref[...])
        out_ref[...] = jnp.dot(x, w3_ref[...],
                               preferred_element_type=jnp.float32) + b3_ref[...]

    full = lambda shape: pl.BlockSpec(shape, lambda i: (0,) * len(shape))
    return pl.pallas_call(
        body,
        grid=grid,
        in_specs=[
            pl.BlockSpec((block_n, d), lambda i: (i, 0)),
            pl.BlockSpec((block_n, de), lambda i: (i, 0)),
            full(na.shape), full(nb.shape), full(b0.shape), full(w1.shape),
            full(b1.shape), full(w2.shape), full(b2.shape), full(w3.shape),
            full(b3.shape),
        ],
        out_specs=pl.BlockSpec((block_n, d), lambda i: (i, 0)),
        out_shape=jax.ShapeDtypeStruct((n, w3.shape[1]), jnp.float32),
    )(h, msg, na, nb, b0, w1, b1, w2, b2, w3, b3)


def kernel(h, edge_index, edge_attr, ew0, ew1, ew2, ew3, ew4, ew5, ew6, ew7,
           nw0, nw1, nw2, nw3, nw4, nw5, nw6, nw7):
    n, d = h.shape
    e, de = edge_attr.shape
    row = edge_index[0]
    col = edge_index[1]

    wa = ew0[:d]
    wb = ew0[d:2 * d]
    w0c = ew0[2 * d:]
    r1 = lambda v: v.reshape(1, -1)

    a, b = _tc_ab(h, wa, wb)
    g = _sc_gather_sum(a, b, row, col)
    ev = _tc_edge_mlp(g, edge_attr, w0c, r1(ew1), ew2, r1(ew3), ew4, r1(ew5),
                      ew6, r1(ew7), block_e=2560)
    partial, half = _sc_scatter_add(ev, col, n)
    partials = partial.reshape(NW, n * de)
    msg = _tc_reduce_partials(partials, block=32000).reshape(n, de)
    h_out = _tc_node_mlp(h, msg, nw0[:d], nw0[d:], r1(nw1), nw2,
                         r1(nw3), nw4, r1(nw5), nw6, r1(nw7), block_n=2000)
    return h_out, ev


# speedup vs baseline: 2.3069x; 1.0011x over previous
import functools

import jax
import jax.numpy as jnp
from jax import lax
from jax.experimental import pallas as pl
from jax.experimental.pallas import tpu as pltpu
from jax.experimental.pallas import tpu_sc as plsc

NC = 2
NS = 16
NW = NC * NS
LANES = 16
CH = 128


def _relu(x):
    return jnp.maximum(x, 0.0)


def _tc_ab(h, wa, wb):
    def body(h_ref, wa_ref, wb_ref, a_ref, b_ref):
        hv = h_ref[...]
        a_ref[...] = jnp.dot(hv, wa_ref[...], preferred_element_type=jnp.float32)
        b_ref[...] = jnp.dot(hv, wb_ref[...], preferred_element_type=jnp.float32)

    n, d = h.shape
    return pl.pallas_call(
        body,
        out_shape=(
            jax.ShapeDtypeStruct((n, wa.shape[1]), jnp.float32),
            jax.ShapeDtypeStruct((n, wb.shape[1]), jnp.float32),
        ),
    )(h, wa, wb)


def _sc_gather_sum(a, b, row, col):
    n, d = a.shape
    e = row.shape[0]
    ew = e // NW
    n_full = ew // CH
    rem = ew % CH
    mesh = plsc.VectorSubcoreMesh(
        core_axis_name="c", subcore_axis_name="s", num_cores=NC, num_subcores=NS
    )

    scratch = [
        pltpu.VMEM((CH,), jnp.int32),
        pltpu.VMEM((CH,), jnp.int32),
        pltpu.VMEM((CH, d), jnp.float32),
        pltpu.VMEM((CH, d), jnp.float32),
        pltpu.SemaphoreType.DMA,
        pltpu.SemaphoreType.DMA,
    ]
    if rem:
        scratch += [
            pltpu.VMEM((rem,), jnp.int32),
            pltpu.VMEM((rem,), jnp.int32),
            pltpu.VMEM((rem, d), jnp.float32),
            pltpu.VMEM((rem, d), jnp.float32),
        ]

    @functools.partial(
        pl.kernel,
        out_type=jax.ShapeDtypeStruct((e, d), jnp.float32),
        mesh=mesh,
        scratch_types=scratch,
    )
    def gather_sum(a_hbm, b_hbm, row_hbm, col_hbm, g_hbm, *scr):
        if rem:
            idxr, idxc, bufa, bufb, sem1, sem2, idxr2, idxc2, bufa2, bufb2 = scr
        else:
            idxr, idxc, bufa, bufb, sem1, sem2 = scr
        wid = lax.axis_index("c") * NS + lax.axis_index("s")
        base = wid * ew

        def do_chunk(off, ch, ir, ic, ba, bb):
            pltpu.sync_copy(row_hbm.at[pl.ds(off, ch)], ir)
            pltpu.sync_copy(col_hbm.at[pl.ds(off, ch)], ic)
            cpa = pltpu.async_copy(a_hbm.at[ir], ba, sem1)
            cpb = pltpu.async_copy(b_hbm.at[ic], bb, sem2)
            cpa.wait()
            cpb.wait()

            def add_row(j, carry):
                for cc in range(d // LANES):
                    sl = pl.ds(cc * LANES, LANES)
                    ba[j, sl] = ba[j, sl] + bb[j, sl]
                return carry

            lax.fori_loop(0, ch, add_row, 0)
            pltpu.sync_copy(ba, g_hbm.at[pl.ds(off, ch)])

        if n_full:
            def step(t, carry):
                do_chunk(base + t * CH, CH, idxr, idxc, bufa, bufb)
                return carry

            lax.fori_loop(0, n_full, step, 0)
        if rem:
            do_chunk(base + n_full * CH, rem, idxr2, idxc2, bufa2, bufb2)

    return gather_sum(a, b, row, col)


def _tc_edge_mlp(g, ea, w0c, b0, w1, b1, w2, b2, w3, b3, block_e):
    e, d = g.shape
    de = ea.shape[1]
    grid = (e // block_e,)

    bf = jnp.bfloat16

    def body(g_ref, ea_ref, w0c_ref, b0_ref, w1_ref, b1_ref, w2_ref, b2_ref,
             w3_ref, b3_ref, out_ref):
        x = g_ref[...] + jnp.dot(ea_ref[...], w0c_ref[...],
                                 preferred_element_type=jnp.float32) + b0_ref[...]
        x = _relu(x)
        x = _relu(jnp.dot(x.astype(bf), w1_ref[...].astype(bf),
                          preferred_element_type=jnp.float32) + b1_ref[...])
        x = _relu(jnp.dot(x.astype(bf), w2_ref[...].astype(bf),
                          preferred_element_type=jnp.float32) + b2_ref[...])
        out_ref[...] = jnp.dot(x.astype(bf), w3_ref[...].astype(bf),
                               preferred_element_type=jnp.float32) + b3_ref[...]

    full = lambda shape: pl.BlockSpec(shape, lambda i: (0,) * len(shape))
    return pl.pallas_call(
        body,
        grid=grid,
        in_specs=[
            pl.BlockSpec((block_e, d), lambda i: (i, 0)),
            pl.BlockSpec((block_e, de), lambda i: (i, 0)),
            full(w0c.shape), full(b0.shape), full(w1.shape), full(b1.shape),
            full(w2.shape), full(b2.shape), full(w3.shape), full(b3.shape),
        ],
        out_specs=pl.BlockSpec((block_e, de), lambda i: (i, 0)),
        out_shape=jax.ShapeDtypeStruct((e, de), jnp.float32),
    )(g, ea, w0c, b0, w1, b1, w2, b2, w3, b3)


def _sc_scatter_add(ev, col, n):
    e, de = ev.shape
    ew = e // NW
    n_full = ew // CH
    rem = ew % CH
    half = n // 2
    mesh = plsc.VectorSubcoreMesh(
        core_axis_name="c", subcore_axis_name="s", num_cores=NC, num_subcores=NS
    )

    scratch = [
        pltpu.VMEM((CH,), jnp.int32),
        pltpu.VMEM((CH, de), jnp.float32),
        pltpu.VMEM((half * de + de,), jnp.float32),
    ]
    if rem:
        scratch += [
            pltpu.VMEM((rem,), jnp.int32),
            pltpu.VMEM((rem, de), jnp.float32),
        ]

    @functools.partial(
        pl.kernel,
        out_type=jax.ShapeDtypeStruct((NW * 2 * half * de,), jnp.float32),
        mesh=mesh,
        scratch_types=scratch,
    )
    def scatter(e_hbm, col_hbm, out_hbm, *scr):
        if rem:
            idxv, ebuf, acc, idxv2, ebuf2 = scr
        else:
            idxv, ebuf, acc = scr
        c = lax.axis_index("c")
        s = lax.axis_index("s")
        wid = c * NS + s
        base = wid * ew
        lanes = lax.iota(jnp.int32, de)

        for p in range(2):
            lo = p * half

            def zrow(j, carry):
                acc[pl.ds(j * de, de)] = jnp.zeros((de,), jnp.float32)
                return carry

            lax.fori_loop(0, half, zrow, 0)

            def do_chunk(off, ch, iv, eb):
                pltpu.sync_copy(col_hbm.at[pl.ds(off, ch)], iv)
                pltpu.sync_copy(e_hbm.at[pl.ds(off, ch)], eb)

                def group(g, carry):
                    colvec = iv[pl.ds(g * LANES, LANES)] * de
                    for l in range(LANES):
                        el = colvec[l] - lo * de
                        ok = jnp.logical_and(el >= 0, el < half * de)
                        off_el = jnp.where(ok, el, half * de)
                        sl = pl.ds(off_el, de)
                        acc[sl] = acc[sl] + eb[g * LANES + l, :]
                    return carry

                lax.fori_loop(0, ch // LANES, group, 0)

            if n_full:
                def step(t, carry):
                    do_chunk(base + t * CH, CH, idxv, ebuf)
                    return carry

                lax.fori_loop(0, n_full, step, 0)
            if rem:
                do_chunk(base + n_full * CH, rem, idxv2, ebuf2)

            pltpu.sync_copy(
                acc.at[pl.ds(0, half * de)],
                out_hbm.at[pl.ds((wid * 2 + p) * half * de, half * de)],
            )

    return scatter(ev, col), half


def _tc_reduce_partials(partials, block):
    nw, tot = partials.shape

    def body(p_ref, out_ref):
        out_ref[...] = jnp.sum(p_ref[...], axis=0)

    return pl.pallas_call(
        body,
        out_shape=jax.ShapeDtypeStruct((tot,), jnp.float32),
    )(partials)


def _tc_node_mlp(h, msg, na, nb, b0, w1, b1, w2, b2, w3, b3, block_n):
    n, d = h.shape
    de = msg.shape[1]
    grid = (n // block_n,)

    def body(h_ref, msg_ref, na_ref, nb_ref, b0_ref, w1_ref, b1_ref,
             w2_ref, b2_ref, w3_ref, b3_ref, out_ref):
        msg = msg_ref[...]
        x = (jnp.dot(h_ref[...], na_ref[...], preferred_element_type=jnp.float32)
             + jnp.dot(msg, nb_ref[...], preferred_element_type=jnp.float32)
             + b0_ref[...])
        x = _relu(x)
        x = _relu(jnp.dot(x, w1_ref[...], preferred_element_type=jnp.float32)
                  + b1_ref[...])
        x = _relu(jnp.dot(x, w2_ref[...], preferred_element_type=jnp.float32)
                  + b2_ref[...])
        out_ref[...] = jnp.dot(x, w3_ref[...],
                               preferred_element_type=jnp.float32) + b3_ref[...]

    full = lambda shape: pl.BlockSpec(shape, lambda i: (0,) * len(shape))
    return pl.pallas_call(
        body,
        grid=grid,
        in_specs=[
            pl.BlockSpec((block_n, d), lambda i: (i, 0)),
            pl.BlockSpec((block_n, de), lambda i: (i, 0)),
            full(na.shape), full(nb.shape), full(b0.shape), full(w1.shape),
            full(b1.shape), full(w2.shape), full(b2.shape), full(w3.shape),
            full(b3.shape),
        ],
        out_specs=pl.BlockSpec((block_n, d), lambda i: (i, 0)),
        out_shape=jax.ShapeDtypeStruct((n, w3.shape[1]), jnp.float32),
    )(h, msg, na, nb, b0, w1, b1, w2, b2, w3, b3)


def kernel(h, edge_index, edge_attr, ew0, ew1, ew2, ew3, ew4, ew5, ew6, ew7,
           nw0, nw1, nw2, nw3, nw4, nw5, nw6, nw7):
    n, d = h.shape
    e, de = edge_attr.shape
    row = edge_index[0]
    col = edge_index[1]

    wa = ew0[:d]
    wb = ew0[d:2 * d]
    w0c = ew0[2 * d:]
    r1 = lambda v: v.reshape(1, -1)

    a, b = _tc_ab(h, wa, wb)
    g = _sc_gather_sum(a, b, row, col)
    ev = _tc_edge_mlp(g, edge_attr, w0c, r1(ew1), ew2, r1(ew3), ew4, r1(ew5),
                      ew6, r1(ew7), block_e=2560)
    partial, half = _sc_scatter_add(ev, col, n)
    partials = partial.reshape(NW, n * de)
    msg = _tc_reduce_partials(partials, block=32000).reshape(n, de)
    h_out = _tc_node_mlp(h, msg, nw0[:d], nw0[d:], r1(nw1), nw2,
                         r1(nw3), nw4, r1(nw5), nw6, r1(nw7), block_n=2000)
    return h_out, ev


# speedup vs baseline: 3.3148x; 1.4369x over previous
import functools

import jax
import jax.numpy as jnp
from jax import lax
from jax.experimental import pallas as pl
from jax.experimental.pallas import tpu as pltpu
from jax.experimental.pallas import tpu_sc as plsc

NC = 2
NS = 16
NW = NC * NS
LANES = 16
CH = 128


def _relu(x):
    return jnp.maximum(x, 0.0)


def _tc_ab(h, wa, wb):
    def body(h_ref, wa_ref, wb_ref, a_ref, b_ref):
        hv = h_ref[...]
        a_ref[...] = jnp.dot(hv, wa_ref[...], preferred_element_type=jnp.float32)
        b_ref[...] = jnp.dot(hv, wb_ref[...], preferred_element_type=jnp.float32)

    n, d = h.shape
    return pl.pallas_call(
        body,
        out_shape=(
            jax.ShapeDtypeStruct((n, wa.shape[1]), jnp.float32),
            jax.ShapeDtypeStruct((n, wb.shape[1]), jnp.float32),
        ),
    )(h, wa, wb)


def _sc_gather_sum(a, b, row_pad, col_pad, e):
    n, d = a.shape
    ew = e // NW
    n_full = ew // CH
    rem = ew % CH
    nchunk = n_full + 1
    ivlen = nchunk * CH
    npairs = (n_full - 2) // 2
    mesh = plsc.VectorSubcoreMesh(
        core_axis_name="c", subcore_axis_name="s", num_cores=NC, num_subcores=NS
    )

    scratch = [
        pltpu.VMEM((ivlen,), jnp.int32),
        pltpu.VMEM((ivlen,), jnp.int32),
        pltpu.VMEM((CH, d), jnp.float32),
        pltpu.VMEM((CH, d), jnp.float32),
        pltpu.VMEM((CH, d), jnp.float32),
        pltpu.VMEM((CH, d), jnp.float32),
        pltpu.SemaphoreType.DMA,
        pltpu.SemaphoreType.DMA,
        pltpu.SemaphoreType.DMA,
        pltpu.SemaphoreType.DMA,
    ]

    @functools.partial(
        pl.kernel,
        out_type=jax.ShapeDtypeStruct((e, d), jnp.float32),
        mesh=mesh,
        scratch_types=scratch,
    )
    def gather_sum(a_hbm, b_hbm, row_hbm, col_hbm, g_hbm, ivr, ivc,
                   ba0, bb0, ba1, bb1, semg0, semg1, sems0, sems1):
        wid = lax.axis_index("c") * NS + lax.axis_index("s")
        base = wid * ew
        pltpu.sync_copy(row_hbm.at[pl.ds(base, ivlen)], ivr)
        pltpu.sync_copy(col_hbm.at[pl.ds(base, ivlen)], ivc)

        slots = ((ba0, bb0, semg0, sems0), (ba1, bb1, semg1, sems1))

        def issue(t, slot):
            ba, bb, semg, _ = slot
            pltpu.async_copy(a_hbm.at[ivr.at[pl.ds(t * CH, CH)]], ba, semg)
            pltpu.async_copy(b_hbm.at[ivc.at[pl.ds(t * CH, CH)]], bb, semg)

        def wait_gathers(slot):
            ba, bb, semg, _ = slot
            pltpu.make_async_copy(a_hbm.at[ivr.at[pl.ds(0, CH)]], ba, semg).wait()
            pltpu.make_async_copy(b_hbm.at[ivc.at[pl.ds(0, CH)]], bb, semg).wait()

        def add_rows(slot, ch):
            ba, bb, _, _ = slot

            def add_row(j, carry):
                for cc in range(d // LANES):
                    sl = pl.ds(cc * LANES, LANES)
                    ba[j, sl] = ba[j, sl] + bb[j, sl]
                return carry

            lax.fori_loop(0, ch, add_row, 0)

        def store_async(t, slot):
            ba, _, _, sems = slot
            pltpu.async_copy(ba, g_hbm.at[pl.ds(base + t * CH, CH)], sems)

        def wait_store(slot):
            ba, _, _, sems = slot
            pltpu.make_async_copy(ba, g_hbm.at[pl.ds(base, CH)], sems).wait()

        issue(0, slots[0])
        issue(1, slots[1])

        def pair(t2, carry):
            t = t2 * 2
            wait_gathers(slots[0])
            add_rows(slots[0], CH)
            store_async(t, slots[0])
            wait_gathers(slots[1])
            add_rows(slots[1], CH)
            store_async(t + 1, slots[1])
            wait_store(slots[0])
            issue(t + 2, slots[0])
            wait_store(slots[1])
            issue(t + 3, slots[1])
            return carry

        lax.fori_loop(0, npairs, pair, 0)

        t = n_full - 2
        wait_gathers(slots[0])
        add_rows(slots[0], CH)
        store_async(t, slots[0])
        wait_gathers(slots[1])
        add_rows(slots[1], CH)
        store_async(t + 1, slots[1])
        wait_store(slots[0])
        issue(n_full, slots[0])
        wait_gathers(slots[0])
        add_rows(slots[0], rem)
        pltpu.sync_copy(
            ba0.at[pl.ds(0, rem)],
            g_hbm.at[pl.ds(base + n_full * CH, rem)],
        )
        wait_store(slots[1])

    return gather_sum(a, b, row_pad, col_pad)


def _tc_edge_mlp(g, ea, w0c, b0, w1, b1, w2, b2, w3, b3, block_e):
    e, d = g.shape
    de = ea.shape[1]
    grid = (e // block_e,)

    bf = jnp.bfloat16

    def body(g_ref, ea_ref, w0c_ref, b0_ref, w1_ref, b1_ref, w2_ref, b2_ref,
             w3_ref, b3_ref, out_ref):
        x = g_ref[...] + jnp.dot(ea_ref[...], w0c_ref[...],
                                 preferred_element_type=jnp.float32) + b0_ref[...]
        x = _relu(x)
        x = _relu(jnp.dot(x.astype(bf), w1_ref[...].astype(bf),
                          preferred_element_type=jnp.float32) + b1_ref[...])
        x = _relu(jnp.dot(x.astype(bf), w2_ref[...].astype(bf),
                          preferred_element_type=jnp.float32) + b2_ref[...])
        out_ref[...] = jnp.dot(x.astype(bf), w3_ref[...].astype(bf),
                               preferred_element_type=jnp.float32) + b3_ref[...]

    full = lambda shape: pl.BlockSpec(shape, lambda i: (0,) * len(shape))
    return pl.pallas_call(
        body,
        grid=grid,
        in_specs=[
            pl.BlockSpec((block_e, d), lambda i: (i, 0)),
            pl.BlockSpec((block_e, de), lambda i: (i, 0)),
            full(w0c.shape), full(b0.shape), full(w1.shape), full(b1.shape),
            full(w2.shape), full(b2.shape), full(w3.shape), full(b3.shape),
        ],
        out_specs=pl.BlockSpec((block_e, de), lambda i: (i, 0)),
        out_shape=jax.ShapeDtypeStruct((e, de), jnp.float32),
    )(g, ea, w0c, b0, w1, b1, w2, b2, w3, b3)


def _sc_scatter_add(ev, col_pad, n):
    e, de = ev.shape
    ew = e // NW
    n_full = ew // CH
    rem = ew % CH
    half = n // 2
    ivlen = n_full * CH + CH
    npairs = (n_full - 2) // 2
    mesh = plsc.VectorSubcoreMesh(
        core_axis_name="c", subcore_axis_name="s", num_cores=NC, num_subcores=NS
    )

    scratch = [
        pltpu.VMEM((ivlen,), jnp.int32),
        pltpu.VMEM((CH, de), jnp.float32),
        pltpu.VMEM((CH, de), jnp.float32),
        pltpu.VMEM((rem, de), jnp.float32),
        pltpu.VMEM((half * de + de,), jnp.float32),
        pltpu.SemaphoreType.DMA,
        pltpu.SemaphoreType.DMA,
    ]

    @functools.partial(
        pl.kernel,
        out_type=jax.ShapeDtypeStruct((NW * 2 * half * de,), jnp.float32),
        mesh=mesh,
        scratch_types=scratch,
    )
    def scatter(e_hbm, col_hbm, out_hbm, iv, eb0, eb1, ebt, acc, sem0, sem1):
        c = lax.axis_index("c")
        s = lax.axis_index("s")
        wid = c * NS + s
        base = wid * ew
        pltpu.sync_copy(col_hbm.at[pl.ds(base, ivlen)], iv)
        slots = ((eb0, sem0), (eb1, sem1))

        def issue(t, slot):
            eb, sem = slot
            pltpu.async_copy(e_hbm.at[pl.ds(base + t * CH, CH)], eb, sem)

        def wait_chunk(slot):
            eb, sem = slot
            pltpu.make_async_copy(e_hbm.at[pl.ds(base, CH)], eb, sem).wait()

        for p in range(2):
            lo = p * half

            def zrow(j, carry):
                acc[pl.ds(j * de, de)] = jnp.zeros((de,), jnp.float32)
                return carry

            lax.fori_loop(0, half, zrow, 0)

            def process(t, eb, ngroups):
                def group(g, carry):
                    colvec = iv[pl.ds(t * CH + g * LANES, LANES)] * de
                    for l in range(LANES):
                        el = colvec[l] - lo * de
                        ok = jnp.logical_and(el >= 0, el < half * de)
                        off_el = jnp.where(ok, el, half * de)
                        sl = pl.ds(off_el, de)
                        acc[sl] = acc[sl] + eb[g * LANES + l, :]
                    return carry

                lax.fori_loop(0, ngroups, group, 0)

            issue(0, slots[0])
            issue(1, slots[1])

            def pair(t2, carry):
                t = t2 * 2
                wait_chunk(slots[0])
                process(t, eb0, CH // LANES)
                issue(t + 2, slots[0])
                wait_chunk(slots[1])
                process(t + 1, eb1, CH // LANES)
                issue(t + 3, slots[1])
                return carry

            lax.fori_loop(0, npairs, pair, 0)
            t = n_full - 2
            wait_chunk(slots[0])
            process(t, eb0, CH // LANES)
            wait_chunk(slots[1])
            process(t + 1, eb1, CH // LANES)
            pltpu.sync_copy(e_hbm.at[pl.ds(base + n_full * CH, rem)], ebt)
            process(n_full, ebt, rem // LANES)

            pltpu.sync_copy(
                acc.at[pl.ds(0, half * de)],
                out_hbm.at[pl.ds((wid * 2 + p) * half * de, half * de)],
            )

    return scatter(ev, col_pad), half


def _tc_reduce_partials(partials, block):
    nw, tot = partials.shape

    def body(p_ref, out_ref):
        out_ref[...] = jnp.sum(p_ref[...], axis=0)

    return pl.pallas_call(
        body,
        out_shape=jax.ShapeDtypeStruct((tot,), jnp.float32),
    )(partials)


def _tc_node_mlp(h, msg, na, nb, b0, w1, b1, w2, b2, w3, b3, block_n):
    n, d = h.shape
    de = msg.shape[1]
    grid = (n // block_n,)

    def body(h_ref, msg_ref, na_ref, nb_ref, b0_ref, w1_ref, b1_ref,
             w2_ref, b2_ref, w3_ref, b3_ref, out_ref):
        msg = msg_ref[...]
        x = (jnp.dot(h_ref[...], na_ref[...], preferred_element_type=jnp.float32)
             + jnp.dot(msg, nb_ref[...], preferred_element_type=jnp.float32)
             + b0_ref[...])
        x = _relu(x)
        x = _relu(jnp.dot(x, w1_ref[...], preferred_element_type=jnp.float32)
                  + b1_ref[...])
        x = _relu(jnp.dot(x, w2_ref[...], preferred_element_type=jnp.float32)
                  + b2_ref[...])
        out_ref[...] = jnp.dot(x, w3_ref[...],
                               preferred_element_type=jnp.float32) + b3_ref[...]

    full = lambda shape: pl.BlockSpec(shape, lambda i: (0,) * len(shape))
    return pl.pallas_call(
        body,
        grid=grid,
        in_specs=[
            pl.BlockSpec((block_n, d), lambda i: (i, 0)),
            pl.BlockSpec((block_n, de), lambda i: (i, 0)),
            full(na.shape), full(nb.shape), full(b0.shape), full(w1.shape),
            full(b1.shape), full(w2.shape), full(b2.shape), full(w3.shape),
            full(b3.shape),
        ],
        out_specs=pl.BlockSpec((block_n, d), lambda i: (i, 0)),
        out_shape=jax.ShapeDtypeStruct((n, w3.shape[1]), jnp.float32),
    )(h, msg, na, nb, b0, w1, b1, w2, b2, w3, b3)


def kernel(h, edge_index, edge_attr, ew0, ew1, ew2, ew3, ew4, ew5, ew6, ew7,
           nw0, nw1, nw2, nw3, nw4, nw5, nw6, nw7):
    n, d = h.shape
    e, de = edge_attr.shape
    row = edge_index[0]
    col = edge_index[1]

    wa = ew0[:d]
    wb = ew0[d:2 * d]
    w0c = ew0[2 * d:]
    r1 = lambda v: v.reshape(1, -1)

    pad = jnp.zeros((CH,), jnp.int32)
    row_pad = jnp.concatenate([row, pad])
    col_pad = jnp.concatenate([col, pad])

    a, b = _tc_ab(h, wa, wb)
    g = _sc_gather_sum(a, b, row_pad, col_pad, e)
    ev = _tc_edge_mlp(g, edge_attr, w0c, r1(ew1), ew2, r1(ew3), ew4, r1(ew5),
                      ew6, r1(ew7), block_e=2560)
    partial, half = _sc_scatter_add(ev, col_pad, n)
    partials = partial.reshape(NW, n * de)
    msg = _tc_reduce_partials(partials, block=32000).reshape(n, de)
    h_out = _tc_node_mlp(h, msg, nw0[:d], nw0[d:], r1(nw1), nw2,
                         r1(nw3), nw4, r1(nw5), nw6, r1(nw7), block_n=2000)
    return h_out, ev


# speedup vs baseline: 3.3290x; 1.0043x over previous
import functools

import jax
import jax.numpy as jnp
from jax import lax
from jax.experimental import pallas as pl
from jax.experimental.pallas import tpu as pltpu
from jax.experimental.pallas import tpu_sc as plsc

NC = 2
NS = 16
NW = NC * NS
LANES = 16
CH = 128


def _relu(x):
    return jnp.maximum(x, 0.0)


def _tc_ab(h, wa, wb):
    def body(h_ref, wa_ref, wb_ref, a_ref, b_ref):
        hv = h_ref[...]
        a_ref[...] = jnp.dot(hv, wa_ref[...], preferred_element_type=jnp.float32)
        b_ref[...] = jnp.dot(hv, wb_ref[...], preferred_element_type=jnp.float32)

    n, d = h.shape
    return pl.pallas_call(
        body,
        out_shape=(
            jax.ShapeDtypeStruct((n, wa.shape[1]), jnp.float32),
            jax.ShapeDtypeStruct((n, wb.shape[1]), jnp.float32),
        ),
    )(h, wa, wb)


def _sc_gather_sum(a, b, row_pad, col_pad, e):
    n, d = a.shape
    ew = e // NW
    n_full = ew // CH
    rem = ew % CH
    nchunk = n_full + 1
    ivlen = nchunk * CH
    npairs = (n_full - 2) // 2
    mesh = plsc.VectorSubcoreMesh(
        core_axis_name="c", subcore_axis_name="s", num_cores=NC, num_subcores=NS
    )

    scratch = [
        pltpu.VMEM((ivlen,), jnp.int32),
        pltpu.VMEM((ivlen,), jnp.int32),
        pltpu.VMEM((CH, d), jnp.float32),
        pltpu.VMEM((CH, d), jnp.float32),
        pltpu.VMEM((CH, d), jnp.float32),
        pltpu.VMEM((CH, d), jnp.float32),
        pltpu.SemaphoreType.DMA,
        pltpu.SemaphoreType.DMA,
        pltpu.SemaphoreType.DMA,
        pltpu.SemaphoreType.DMA,
    ]

    @functools.partial(
        pl.kernel,
        out_type=jax.ShapeDtypeStruct((e, d), jnp.float32),
        mesh=mesh,
        scratch_types=scratch,
    )
    def gather_sum(a_hbm, b_hbm, row_hbm, col_hbm, g_hbm, ivr, ivc,
                   ba0, bb0, ba1, bb1, semg0, semg1, sems0, sems1):
        wid = lax.axis_index("c") * NS + lax.axis_index("s")
        base = wid * ew
        pltpu.sync_copy(row_hbm.at[pl.ds(base, ivlen)], ivr)
        pltpu.sync_copy(col_hbm.at[pl.ds(base, ivlen)], ivc)

        slots = ((ba0, bb0, semg0, sems0), (ba1, bb1, semg1, sems1))

        def issue(t, slot):
            ba, bb, semg, _ = slot
            pltpu.async_copy(a_hbm.at[ivr.at[pl.ds(t * CH, CH)]], ba, semg)
            pltpu.async_copy(b_hbm.at[ivc.at[pl.ds(t * CH, CH)]], bb, semg)

        def wait_gathers(slot):
            ba, bb, semg, _ = slot
            pltpu.make_async_copy(a_hbm.at[ivr.at[pl.ds(0, CH)]], ba, semg).wait()
            pltpu.make_async_copy(b_hbm.at[ivc.at[pl.ds(0, CH)]], bb, semg).wait()

        def add_rows(slot, ch):
            ba, bb, _, _ = slot

            def add_row(j, carry):
                for cc in range(d // LANES):
                    sl = pl.ds(cc * LANES, LANES)
                    ba[j, sl] = ba[j, sl] + bb[j, sl]
                return carry

            lax.fori_loop(0, ch, add_row, 0)

        def store_async(t, slot):
            ba, _, _, sems = slot
            pltpu.async_copy(ba, g_hbm.at[pl.ds(base + t * CH, CH)], sems)

        def wait_store(slot):
            ba, _, _, sems = slot
            pltpu.make_async_copy(ba, g_hbm.at[pl.ds(base, CH)], sems).wait()

        issue(0, slots[0])
        issue(1, slots[1])

        def pair(t2, carry):
            t = t2 * 2
            wait_gathers(slots[0])
            add_rows(slots[0], CH)
            store_async(t, slots[0])
            wait_gathers(slots[1])
            add_rows(slots[1], CH)
            store_async(t + 1, slots[1])
            wait_store(slots[0])
            issue(t + 2, slots[0])
            wait_store(slots[1])
            issue(t + 3, slots[1])
            return carry

        lax.fori_loop(0, npairs, pair, 0)

        t = n_full - 2
        wait_gathers(slots[0])
        add_rows(slots[0], CH)
        store_async(t, slots[0])
        wait_gathers(slots[1])
        add_rows(slots[1], CH)
        store_async(t + 1, slots[1])
        wait_store(slots[0])
        issue(n_full, slots[0])
        wait_gathers(slots[0])
        add_rows(slots[0], rem)
        pltpu.sync_copy(
            ba0.at[pl.ds(0, rem)],
            g_hbm.at[pl.ds(base + n_full * CH, rem)],
        )
        wait_store(slots[1])

    return gather_sum(a, b, row_pad, col_pad)


def _tc_edge_mlp(g, ea, w0c, b0, w1, b1, w2, b2, w3, b3, block_e):
    e, d = g.shape
    de = ea.shape[1]
    grid = (e // block_e,)

    bf = jnp.bfloat16

    def body(g_ref, ea_ref, w0c_ref, b0_ref, w1_ref, b1_ref,
             w2_ref, b2_ref, w3_ref, b3_ref, out_ref):
        x = g_ref[...] + jnp.dot(ea_ref[...], w0c_ref[...],
                                 preferred_element_type=jnp.float32) + b0_ref[...]
        x = _relu(x)
        x = _relu(jnp.dot(x.astype(bf), w1_ref[...].astype(bf),
                          preferred_element_type=jnp.float32) + b1_ref[...])
        x = _relu(jnp.dot(x.astype(bf), w2_ref[...].astype(bf),
                          preferred_element_type=jnp.float32) + b2_ref[...])
        out_ref[...] = jnp.dot(x.astype(bf), w3_ref[...].astype(bf),
                               preferred_element_type=jnp.float32) + b3_ref[...]

    full = lambda shape: pl.BlockSpec(shape, lambda i: (0,) * len(shape))
    return pl.pallas_call(
        body,
        grid=grid,
        in_specs=[
            pl.BlockSpec((block_e, d), lambda i: (i, 0)),
            pl.BlockSpec((block_e, de), lambda i: (i, 0)),
            full(w0c.shape), full(b0.shape), full(w1.shape), full(b1.shape),
            full(w2.shape), full(b2.shape), full(w3.shape), full(b3.shape),
        ],
        out_specs=pl.BlockSpec((block_e, de), lambda i: (i, 0)),
        out_shape=jax.ShapeDtypeStruct((e, de), jnp.float32),
    )(g, ea, w0c, b0, w1, b1, w2, b2, w3, b3)


def _sc_scatter_add(ev, col_pad, n):
    e, de = ev.shape
    ew = e // NW
    n_full = ew // CH
    rem = ew % CH
    half = n // 2
    ivlen = n_full * CH + CH
    npairs = (n_full - 2) // 2
    mesh = plsc.VectorSubcoreMesh(
        core_axis_name="c", subcore_axis_name="s", num_cores=NC, num_subcores=NS
    )

    scratch = [
        pltpu.VMEM((ivlen,), jnp.int32),
        pltpu.VMEM((CH, de), jnp.float32),
        pltpu.VMEM((CH, de), jnp.float32),
        pltpu.VMEM((rem, de), jnp.float32),
        pltpu.VMEM((half * de + de,), jnp.float32),
        pltpu.SemaphoreType.DMA,
        pltpu.SemaphoreType.DMA,
    ]

    @functools.partial(
        pl.kernel,
        out_type=jax.ShapeDtypeStruct((NW * 2 * half * de,), jnp.float32),
        mesh=mesh,
        scratch_types=scratch,
    )
    def scatter(e_hbm, col_hbm, out_hbm, iv, eb0, eb1, ebt, acc, sem0, sem1):
        c = lax.axis_index("c")
        s = lax.axis_index("s")
        wid = c * NS + s
        base = wid * ew
        pltpu.sync_copy(col_hbm.at[pl.ds(base, ivlen)], iv)
        slots = ((eb0, sem0), (eb1, sem1))

        def issue(t, slot):
            eb, sem = slot
            pltpu.async_copy(e_hbm.at[pl.ds(base + t * CH, CH)], eb, sem)

        def wait_chunk(slot):
            eb, sem = slot
            pltpu.make_async_copy(e_hbm.at[pl.ds(base, CH)], eb, sem).wait()

        for p in range(2):
            lo = p * half

            def zrow(j, carry):
                acc[pl.ds(j * de, de)] = jnp.zeros((de,), jnp.float32)
                return carry

            lax.fori_loop(0, half, zrow, 0)

            def process(t, eb, ngroups):
                def group(g, carry):
                    colvec = iv[pl.ds(t * CH + g * LANES, LANES)] * de
                    for l in range(LANES):
                        el = colvec[l] - lo * de
                        ok = jnp.logical_and(el >= 0, el < half * de)
                        off_el = jnp.where(ok, el, half * de)
                        sl = pl.ds(off_el, de)
                        acc[sl] = acc[sl] + eb[g * LANES + l, :]
                    return carry

                lax.fori_loop(0, ngroups, group, 0)

            issue(0, slots[0])
            issue(1, slots[1])

            def pair(t2, carry):
                t = t2 * 2
                wait_chunk(slots[0])
                process(t, eb0, CH // LANES)
                issue(t + 2, slots[0])
                wait_chunk(slots[1])
                process(t + 1, eb1, CH // LANES)
                issue(t + 3, slots[1])
                return carry

            lax.fori_loop(0, npairs, pair, 0)
            t = n_full - 2
            wait_chunk(slots[0])
            process(t, eb0, CH // LANES)
            wait_chunk(slots[1])
            process(t + 1, eb1, CH // LANES)
            pltpu.sync_copy(e_hbm.at[pl.ds(base + n_full * CH, rem)], ebt)
            process(n_full, ebt, rem // LANES)

            pltpu.sync_copy(
                acc.at[pl.ds(0, half * de)],
                out_hbm.at[pl.ds((wid * 2 + p) * half * de, half * de)],
            )

    return scatter(ev, col_pad), half


def _tc_reduce_partials(partials, block):
    nw, tot = partials.shape

    def body(p_ref, out_ref):
        out_ref[...] = jnp.sum(p_ref[...], axis=0)

    return pl.pallas_call(
        body,
        out_shape=jax.ShapeDtypeStruct((tot,), jnp.float32),
    )(partials)


def _tc_node_mlp(h, msg, na, nb, b0, w1, b1, w2, b2, w3, b3, block_n):
    n, d = h.shape
    de = msg.shape[1]
    grid = (n // block_n,)

    def body(h_ref, msg_ref, na_ref, nb_ref, b0_ref, w1_ref, b1_ref,
             w2_ref, b2_ref, w3_ref, b3_ref, out_ref):
        msg = msg_ref[...]
        x = (jnp.dot(h_ref[...], na_ref[...], preferred_element_type=jnp.float32)
             + jnp.dot(msg, nb_ref[...], preferred_element_type=jnp.float32)
             + b0_ref[...])
        x = _relu(x)
        x = _relu(jnp.dot(x, w1_ref[...], preferred_element_type=jnp.float32)
                  + b1_ref[...])
        x = _relu(jnp.dot(x, w2_ref[...], preferred_element_type=jnp.float32)
                  + b2_ref[...])
        out_ref[...] = jnp.dot(x, w3_ref[...],
                               preferred_element_type=jnp.float32) + b3_ref[...]

    full = lambda shape: pl.BlockSpec(shape, lambda i: (0,) * len(shape))
    return pl.pallas_call(
        body,
        grid=grid,
        in_specs=[
            pl.BlockSpec((block_n, d), lambda i: (i, 0)),
            pl.BlockSpec((block_n, de), lambda i: (i, 0)),
            full(na.shape), full(nb.shape), full(b0.shape), full(w1.shape),
            full(b1.shape), full(w2.shape), full(b2.shape), full(w3.shape),
            full(b3.shape),
        ],
        out_specs=pl.BlockSpec((block_n, d), lambda i: (i, 0)),
        out_shape=jax.ShapeDtypeStruct((n, w3.shape[1]), jnp.float32),
    )(h, msg, na, nb, b0, w1, b1, w2, b2, w3, b3)


def kernel(h, edge_index, edge_attr, ew0, ew1, ew2, ew3, ew4, ew5, ew6, ew7,
           nw0, nw1, nw2, nw3, nw4, nw5, nw6, nw7):
    n, d = h.shape
    e, de = edge_attr.shape
    row = edge_index[0]
    col = edge_index[1]

    wa = ew0[:d]
    wb = ew0[d:2 * d]
    w0c = ew0[2 * d:]
    r1 = lambda v: v.reshape(1, -1)

    pad = jnp.zeros((CH,), jnp.int32)
    row_pad = jnp.concatenate([row, pad])
    col_pad = jnp.concatenate([col, pad])

    a, b = _tc_ab(h, wa, wb)
    g = _sc_gather_sum(a, b, row_pad, col_pad, e)
    ev = _tc_edge_mlp(g, edge_attr, w0c, r1(ew1), ew2, r1(ew3), ew4,
                      r1(ew5), ew6, r1(ew7), block_e=5000)
    partial, half = _sc_scatter_add(ev, col_pad, n)
    partials = partial.reshape(NW, n * de)
    msg = _tc_reduce_partials(partials, block=32000).reshape(n, de)
    h_out = _tc_node_mlp(h, msg, nw0[:d], nw0[d:], r1(nw1), nw2,
                         r1(nw3), nw4, r1(nw5), nw6, r1(nw7), block_n=2000)
    return h_out, ev
